# baseline (speedup 1.0000x reference)
import functools
import math

import jax
import jax.numpy as jnp
from jax import lax
from jax.experimental import pallas as pl
from jax.experimental.pallas import tpu as pltpu
from jax.experimental.pallas import tpu_sc as plsc

_L = 16

_ATAN_C = (
    0.99999612,
    -0.33317369,
    0.19807802,
    -0.13233266,
    0.07962221,
    -0.03360298,
    0.0068114,
)
_HALF_PI = math.pi / 2


def _atan_ratio(num, den):
    an = jnp.abs(num)
    ad = jnp.abs(den)
    t = jnp.minimum(an, ad) / jnp.maximum(an, ad)
    z = t * t
    p = jnp.float32(_ATAN_C[-1])
    for c in _ATAN_C[-2::-1]:
        p = p * z + jnp.float32(c)
    p = p * t
    r = jnp.where(an > ad, _HALF_PI - p, p)
    return jnp.where(num * den < 0.0, -r, r)


def _make_sc_kernel(n, num_classes):
    info = plsc.get_sparse_core_info()
    nc, ns = info.num_cores, info.num_subcores
    nw = nc * ns
    assert n % (nw * _L) == 0
    bpw = n // nw
    groups = bpw // _L
    chunk = 128
    nchunks = bpw // chunk
    per_chunk = chunk // _L

    mesh = plsc.VectorSubcoreMesh(core_axis_name="c", subcore_axis_name="s")

    @functools.partial(
        pl.kernel,
        mesh=mesh,
        compiler_params=pltpu.CompilerParams(
            needs_layout_passes=False, use_tc_tiling_on_sc=False),
        out_type=[
            jax.ShapeDtypeStruct((n,), jnp.float32),
            jax.ShapeDtypeStruct((n // 128, 4, 128), jnp.float32),
            jax.ShapeDtypeStruct((n // 128, 8, 128), jnp.float32),
            jax.ShapeDtypeStruct((n,), jnp.float32),
        ],
        scratch_types=[
            pltpu.VMEM((bpw,), jnp.int32),
            pltpu.VMEM((nchunks, chunk), jnp.int32),
            pltpu.VMEM((num_classes // 8, bpw // 128, 8, 128),
                       jnp.float32),
            pltpu.VMEM((2, num_classes * 3 // 8, 8, 128),
                       jnp.float32),
            pltpu.VMEM((bpw // 128, 4, 128), jnp.float32),
            pltpu.VMEM((bpw, 8), jnp.float32),
            pltpu.VMEM((bpw // 128, 8, 128), jnp.float32),
            pltpu.VMEM((bpw,), jnp.float32),
            pltpu.VMEM((bpw,), jnp.float32),
            pltpu.SemaphoreType.DMA((bpw // 128,)),
            pltpu.SemaphoreType.DMA,
            pltpu.SemaphoreType.DMA,
        ],
    )
    def sc_kernel(dep_hbm, dims_hbm, rots_hbm, lab_hbm,
                  d_out, dims_out, rots_out, alpha_out,
                  lab_v, idxr_v, dep_slab, dims_slab, dcol_v, rots_v,
                  rcomp_v, dout_v, alph_v, gsem, osem, dsem):
        wid = lax.axis_index("s") * nc + lax.axis_index("c")
        base = wid * bpw
        blk = base // 128
        dep_copies = [
            pltpu.async_copy(
                dep_hbm.at[a].at[pl.ds(blk, bpw // 128)],
                dep_slab.at[a], dsem)
            for a in range(num_classes // 8)
        ]
        pltpu.sync_copy(lab_hbm.at[pl.ds(base, bpw)], lab_v)

        iota = lax.iota(jnp.int32, _L)

        def build_idx(g):
            off = g * _L
            c = g // per_chunk
            k = off - c * chunk
            lab = lab_v[pl.ds(off, _L)]
            row = (base + off) + iota
            idxr_v[c, pl.ds(k, _L)] = row * num_classes + lab

        def fire_dims(c):
            return [pltpu.async_copy(
                dims_hbm.at[j].at[pl.ds(blk + c, 1)],
                dims_slab.at[c % 2].at[pl.ds(j, 1)],
                gsem.at[c]) for j in range(num_classes * 3 // 8)]

        dims_copies = {c: fire_dims(c) for c in range(2)}
        copies = []
        for c in range(nchunks):
            plsc.parallel_loop(
                c * per_chunk, (c + 1) * per_chunk, unroll=2)(build_idx)
            o = c * chunk
            copies.append(pltpu.async_copy(
                rots_hbm.at[idxr_v.at[c]], rots_v.at[pl.ds(o, chunk)],
                gsem.at[c]))

        def compute(g):
            off = g * _L
            c = g // per_chunk
            k = off - c * chunk
            lab = lab_v[pl.ds(off, _L)]
            cvec = jnp.zeros((_L,), jnp.int32) + c
            dep = plsc.load_gather(
                dep_slab, [lab >> 3, cvec, lab & 7, k + iota])
            dout_v[pl.ds(off, _L)] = jnp.exp(dep) - 1.0

            buf = jnp.zeros((_L,), jnp.int32) + (c & 1)
            cd = lab * 3
            for j in range(3):
                cdj = cd + j
                dv = plsc.load_gather(
                    dims_slab, [buf, cdj >> 3, cdj & 7, k + iota])
                dcol_v[c, j, pl.ds(k, _L)] = dv

            rows = off + iota
            r = []
            for j in range(8):
                cj = jnp.full((_L,), j, jnp.int32)
                v = plsc.load_gather(rots_v, [rows, cj])
                rcomp_v[c, j, pl.ds(k, _L)] = v
                r.append(v)
            a1 = _atan_ratio(r[2], r[3]) - _HALF_PI
            a2 = _atan_ratio(r[6], r[7]) + _HALF_PI
            alph_v[pl.ds(off, _L)] = jnp.where(r[1] > r[5], a1, a2)

        for cp in dep_copies:
            cp.wait()
        for c in range(nchunks):
            copies[c].wait()
            for cp in dims_copies.pop(c):
                cp.wait()
            plsc.parallel_loop(
                c * per_chunk, (c + 1) * per_chunk, unroll=2)(compute)
            if c + 2 < nchunks:
                dims_copies[c + 2] = fire_dims(c + 2)

        blk = base // 128
        out_copies = [
            pltpu.async_copy(dout_v, d_out.at[pl.ds(base, bpw)], osem),
            pltpu.async_copy(alph_v, alpha_out.at[pl.ds(base, bpw)], osem),
            pltpu.async_copy(
                dcol_v, dims_out.at[pl.ds(blk, bpw // 128)], osem),
            pltpu.async_copy(
                rcomp_v, rots_out.at[pl.ds(blk, bpw // 128)], osem),
        ]
        for cp in out_copies:
            cp.wait()

    return sc_kernel


def kernel(depths, dims, rots, labels):
    n, num_classes = depths.shape
    dep_t = (depths.T.reshape(num_classes // 8, 8, n // 128, 128)
             .transpose(0, 2, 1, 3))
    dims_t = (dims.T.reshape(num_classes * 3 // 8, 8, n // 128, 128)
              .transpose(0, 2, 1, 3))
    rots_flat = rots.reshape(n * num_classes, 8)
    lab = labels.astype(jnp.int32)
    d, dims_t4, rots_t8, alphas = _make_sc_kernel(n, num_classes)(
        dep_t, dims_t, rots_flat, lab)
    dims_g = dims_t4[:, :3, :].transpose(0, 2, 1).reshape(n, 3)
    rots_g = rots_t8.transpose(0, 2, 1).reshape(n, 8)
    return d.reshape(n, 1), dims_g, rots_g, alphas


# speedup vs baseline: 1.1091x; 1.1091x over previous
import functools
import math

import jax
import jax.numpy as jnp
from jax import lax
from jax.experimental import pallas as pl
from jax.experimental.pallas import tpu as pltpu
from jax.experimental.pallas import tpu_sc as plsc

_L = 16

_ATAN_C = (
    0.99999612,
    -0.33317369,
    0.19807802,
    -0.13233266,
    0.07962221,
    -0.03360298,
    0.0068114,
)
_HALF_PI = math.pi / 2


def _atan_ratio(num, den):
    an = jnp.abs(num)
    ad = jnp.abs(den)
    t = jnp.minimum(an, ad) / jnp.maximum(an, ad)
    z = t * t
    p = jnp.float32(_ATAN_C[-1])
    for c in _ATAN_C[-2::-1]:
        p = p * z + jnp.float32(c)
    p = p * t
    r = jnp.where(an > ad, _HALF_PI - p, p)
    return jnp.where(num * den < 0.0, -r, r)


def _make_sc_kernel(n, num_classes):
    info = plsc.get_sparse_core_info()
    nc, ns = info.num_cores, info.num_subcores
    nw = nc * ns
    assert n % (nw * _L) == 0
    bpw = n // nw
    groups = bpw // _L
    chunk = 128
    nchunks = bpw // chunk
    per_chunk = chunk // _L

    mesh = plsc.VectorSubcoreMesh(core_axis_name="c", subcore_axis_name="s")

    @functools.partial(
        pl.kernel,
        mesh=mesh,
        compiler_params=pltpu.CompilerParams(
            needs_layout_passes=False, use_tc_tiling_on_sc=False),
        out_type=[
            jax.ShapeDtypeStruct((n,), jnp.float32),
            jax.ShapeDtypeStruct((n // 128, 4, 128), jnp.float32),
            jax.ShapeDtypeStruct((n // 128, 8, 128), jnp.float32),
            jax.ShapeDtypeStruct((n,), jnp.float32),
        ],
        scratch_types=[
            pltpu.VMEM((bpw,), jnp.int32),
            pltpu.VMEM((nchunks, chunk), jnp.int32),
            pltpu.VMEM((3, nchunks, chunk), jnp.int32),
            pltpu.VMEM((num_classes // 8, bpw // 128, 8, 128),
                       jnp.float32),
            pltpu.VMEM((bpw // 128, 4, 128), jnp.float32),
            pltpu.VMEM((bpw, 8), jnp.float32),
            pltpu.VMEM((bpw // 128, 8, 128), jnp.float32),
            pltpu.VMEM((bpw,), jnp.float32),
            pltpu.VMEM((bpw,), jnp.float32),
            pltpu.SemaphoreType.DMA((bpw // 128,)),
            pltpu.SemaphoreType.DMA,
            pltpu.SemaphoreType.DMA,
        ],
    )
    def sc_kernel(dep_hbm, dims_hbm, rots_hbm, lab_hbm,
                  d_out, dims_out, rots_out, alpha_out,
                  lab_v, idxr_v, idx3_v, dep_slab, dcol_v, rots_v,
                  rcomp_v, dout_v, alph_v, gsem, osem, dsem):
        wid = lax.axis_index("s") * nc + lax.axis_index("c")
        base = wid * bpw
        blk = base // 128
        dep_copies = [
            pltpu.async_copy(
                dep_hbm.at[a].at[pl.ds(blk, bpw // 128)],
                dep_slab.at[a], dsem)
            for a in range(num_classes // 8)
        ]
        pltpu.sync_copy(lab_hbm.at[pl.ds(base, bpw)], lab_v)

        iota = lax.iota(jnp.int32, _L)

        def build_idx(g):
            off = g * _L
            c = g // per_chunk
            k = off - c * chunk
            lab = lab_v[pl.ds(off, _L)]
            row = (base + off) + iota
            idxr_v[c, pl.ds(k, _L)] = row * num_classes + lab
            tile = (row >> 7) * 1024 + (row & 127)
            cd = lab * 3
            for j in range(3):
                cdj = cd + j
                idx3_v[j, c, pl.ds(k, _L)] = (
                    (cdj >> 3) * (8 * n) + ((cdj & 7) << 7) + tile)

        copies = []
        for c in range(nchunks):
            plsc.parallel_loop(
                c * per_chunk, (c + 1) * per_chunk, unroll=2)(build_idx)
            o = c * chunk
            sem = gsem.at[c]
            copies.append(pltpu.async_copy(
                rots_hbm.at[idxr_v.at[c]], rots_v.at[pl.ds(o, chunk)], sem))
            for j in range(3):
                copies.append(pltpu.async_copy(
                    dims_hbm.at[idx3_v.at[j].at[c]],
                    dcol_v.at[c].at[j], sem))

        def compute(g):
            off = g * _L
            c = g // per_chunk
            k = off - c * chunk
            lab = lab_v[pl.ds(off, _L)]
            cvec = jnp.zeros((_L,), jnp.int32) + c
            dep = plsc.load_gather(
                dep_slab, [lab >> 3, cvec, lab & 7, k + iota])
            dout_v[pl.ds(off, _L)] = jnp.exp(dep) - 1.0

            rows = off + iota
            r = []
            for j in range(8):
                cj = jnp.full((_L,), j, jnp.int32)
                v = plsc.load_gather(rots_v, [rows, cj])
                rcomp_v[c, j, pl.ds(k, _L)] = v
                r.append(v)
            a1 = _atan_ratio(r[2], r[3]) - _HALF_PI
            a2 = _atan_ratio(r[6], r[7]) + _HALF_PI
            alph_v[pl.ds(off, _L)] = jnp.where(r[1] > r[5], a1, a2)

        for cp in dep_copies:
            cp.wait()
        for c in range(nchunks):
            for cp in copies[4 * c:4 * c + 4]:
                cp.wait()
            plsc.parallel_loop(
                c * per_chunk, (c + 1) * per_chunk, unroll=2)(compute)

        out_copies = [
            pltpu.async_copy(dout_v, d_out.at[pl.ds(base, bpw)], osem),
            pltpu.async_copy(alph_v, alpha_out.at[pl.ds(base, bpw)], osem),
            pltpu.async_copy(
                dcol_v, dims_out.at[pl.ds(blk, bpw // 128)], osem),
            pltpu.async_copy(
                rcomp_v, rots_out.at[pl.ds(blk, bpw // 128)], osem),
        ]
        for cp in out_copies:
            cp.wait()

    return sc_kernel


def kernel(depths, dims, rots, labels):
    n, num_classes = depths.shape
    dep_t = (depths.T.reshape(num_classes // 8, 8, n // 128, 128)
             .transpose(0, 2, 1, 3))
    dims_t = (dims.T.reshape(num_classes * 3 // 8, 8, n // 128, 128)
              .transpose(0, 2, 1, 3).reshape(n * num_classes * 3))
    rots_flat = rots.reshape(n * num_classes, 8)
    lab = labels.astype(jnp.int32)
    d, dims_t4, rots_t8, alphas = _make_sc_kernel(n, num_classes)(
        dep_t, dims_t, rots_flat, lab)
    dims_g = dims_t4[:, :3, :].transpose(0, 2, 1).reshape(n, 3)
    rots_g = rots_t8.transpose(0, 2, 1).reshape(n, 8)
    return d.reshape(n, 1), dims_g, rots_g, alphas
